# baseline (speedup 1.0000x reference)
import functools

import jax
import jax.numpy as jnp
from jax.experimental import pallas as pl
from jax.experimental.pallas import tpu as pltpu


def _conv_body(x_ref, w1_ref, b1_ref, w2_ref, b2_ref, w3_ref, b3_ref,
               o_ref, *, B):
    f32 = jnp.float32
    bf16 = jnp.bfloat16

    X = x_ref[...]
    y = jnp.dot(X, w1_ref[...], preferred_element_type=f32)
    y = y.reshape(B, 21, 16, 128)
    a1 = y[:, 0:20, :, 0:64] + y[:, 1:21, :, 64:128]
    a1 = jnp.maximum(a1 + b1_ref[...], 0.0).astype(bf16)

    arp = a1.reshape(B, 10, 2, 16, 64)
    h1 = jnp.concatenate([arp[:, :, 0], arp[:, :, 1]], axis=-1)
    h1s = jnp.pad(h1[:, :, 1:, :], ((0, 0), (0, 0), (0, 1), (0, 0)))
    X2 = jnp.concatenate([h1, h1s], axis=-1).reshape(B * 160, 256)

    z = jnp.dot(X2, w2_ref[...], preferred_element_type=f32)
    z = z.reshape(B, 10, 16, 128)
    a2 = z[:, 0:9, :, 0:64] + z[:, 1:10, :, 64:128]
    a2 = jnp.maximum(a2 + b2_ref[...], 0.0).astype(bf16)

    h2s1 = jnp.pad(a2[:, :, 1:, :], ((0, 0), (0, 0), (0, 1), (0, 0)))
    h2s2 = jnp.pad(a2[:, :, 2:, :], ((0, 0), (0, 0), (0, 2), (0, 0)))
    X3 = jnp.concatenate([a2, h2s1, h2s2], axis=-1).reshape(B * 144, 192)

    v = jnp.dot(X3, w3_ref[...], preferred_element_type=f32)
    v = v.reshape(B, 9, 16, 192)
    a3 = v[:, 0:7, :, 0:64] + v[:, 1:8, :, 64:128] + v[:, 2:9, :, 128:192]
    a3 = jnp.maximum(a3 + b3_ref[...], 0.0)

    o_ref[...] = a3[:, :, 0:7, :]


def _fc_kernel(x_ref, w0_ref, b0_ref, w1_ref, b1_ref, o_ref):
    f32 = jnp.float32
    h = jnp.dot(x_ref[...], w0_ref[...], preferred_element_type=f32)
    h = jnp.maximum(h + b0_ref[...], 0.0).astype(jnp.bfloat16)
    out = jnp.dot(h, w1_ref[...], preferred_element_type=f32)
    o_ref[...] = out + b1_ref[...]


def kernel(x_nchw, conv_taps_0, conv_taps_1, conv_taps_2,
          conv_bias_0, conv_bias_1, conv_bias_2,
          fc_w_0, fc_w_1, fc_b_0, fc_b_1):
    N = x_nchw.shape[0]
    bf16 = jnp.bfloat16
    B = 32

    w1g = conv_taps_0.reshape(2, 4, 2, 4, 4, 128)[..., :32]
    w1g = w1g.transpose(0, 2, 1, 3, 4, 5)
    w1n = jnp.zeros((4, 4, 3, 4, 2, 2, 32), jnp.float32)
    for a in range(2):
        for pw in range(2):
            for q in range(2):
                wt = w1g[a, q].transpose(0, 2, 1, 3)
                w1n = w1n.at[:, :, pw + q, :, a, pw, :].add(wt)
    w1 = w1n.reshape(192, 128).astype(bf16)
    b1c = jnp.concatenate([conv_bias_0[:, :32]] * 2, axis=1)
    w2 = conv_taps_1.reshape(2, 2, 2, 2, 128, 128)[:, :, :, :, :32, :64]
    w2 = w2.transpose(0, 2, 1, 3, 4, 5).reshape(2, 2, 128, 64)
    w2 = jnp.concatenate([w2[:, 0], w2[:, 1]], axis=1)
    w2 = jnp.concatenate([w2[0], w2[1]], axis=1).astype(bf16)
    w3 = conv_taps_2[:, :64, :64].reshape(3, 3, 64, 64)
    w3 = jnp.concatenate([w3[:, 0], w3[:, 1], w3[:, 2]], axis=1)
    w3 = jnp.concatenate([w3[0], w3[1], w3[2]], axis=1).astype(bf16)
    b2 = conv_bias_1[:, :64]
    b3 = conv_bias_2[:, :64]
    fw0 = fc_w_0.reshape(49, 128, 512)[:, :64, :].reshape(3136, 512)
    fw0 = fw0.astype(bf16)
    fw1 = fc_w_1.astype(bf16)

    xt = x_nchw.transpose(0, 2, 1, 3).astype(bf16)
    xw = jnp.pad(xt, ((0, 0), (0, 0), (0, 0), (0, 44)))
    pa = xw.reshape(N, 21, 4, 4, 16, 8)
    pb = jnp.pad(xw[:, :, :, 8:], ((0, 0), (0, 0), (0, 0), (0, 8)))
    pb = pb.reshape(N, 21, 4, 4, 16, 8)[..., 0:4]
    xo = jnp.concatenate([pa, pb], axis=-1)
    x2 = xo.transpose(0, 1, 4, 2, 3, 5).reshape(N * 336, 192)

    conv_body = functools.partial(_conv_body, B=B)
    feats = pl.pallas_call(
        conv_body,
        out_shape=jax.ShapeDtypeStruct((N, 7, 7, 64), jnp.float32),
        grid=(N // B,),
        in_specs=[
            pl.BlockSpec((B * 336, 192), lambda n: (n, 0)),
            pl.BlockSpec((192, 128), lambda n: (0, 0)),
            pl.BlockSpec((1, 64), lambda n: (0, 0)),
            pl.BlockSpec((256, 128), lambda n: (0, 0)),
            pl.BlockSpec((1, 64), lambda n: (0, 0)),
            pl.BlockSpec((192, 192), lambda n: (0, 0)),
            pl.BlockSpec((1, 64), lambda n: (0, 0)),
        ],
        out_specs=pl.BlockSpec((B, 7, 7, 64), lambda n: (n, 0, 0, 0)),
        compiler_params=pltpu.CompilerParams(
            dimension_semantics=("parallel",)),
    )(x2, w1, b1c, w2, b2, w3, b3)

    flat = feats.reshape(N, 3136).astype(bf16)
    TM = 256 if N >= 256 else N
    out = pl.pallas_call(
        _fc_kernel,
        out_shape=jax.ShapeDtypeStruct((N, 128), jnp.float32),
        grid=(N // TM,),
        in_specs=[
            pl.BlockSpec((TM, 3136), lambda i: (i, 0)),
            pl.BlockSpec((3136, 512), lambda i: (0, 0)),
            pl.BlockSpec((1, 512), lambda i: (0, 0)),
            pl.BlockSpec((512, 128), lambda i: (0, 0)),
            pl.BlockSpec((1, 128), lambda i: (0, 0)),
        ],
        out_specs=pl.BlockSpec((TM, 128), lambda i: (i, 0)),
        compiler_params=pltpu.CompilerParams(
            dimension_semantics=("parallel",)),
    )(flat, fw0, fc_b_0, fw1, fc_b_1)
    return out[:, :18]


# speedup vs baseline: 14.6948x; 14.6948x over previous
import functools

import jax
import jax.numpy as jnp
from jax.experimental import pallas as pl
from jax.experimental.pallas import tpu as pltpu


def _conv_body(x_ref, w1_ref, b1_ref, w2_ref, b2_ref, w3_ref, b3_ref,
               o_ref, *, B):
    f32 = jnp.float32
    bf16 = jnp.bfloat16

    X = x_ref[...]
    y = jnp.dot(X, w1_ref[...], preferred_element_type=f32)
    y = y.reshape(B, 21, 16, 128)
    a1 = y[:, 0:20, :, 0:64] + y[:, 1:21, :, 64:128]
    a1 = jnp.maximum(a1 + b1_ref[...], 0.0).astype(bf16)

    arp = a1.reshape(B, 10, 2, 16, 64)
    h1 = jnp.concatenate([arp[:, :, 0], arp[:, :, 1]], axis=-1)
    h1s = jnp.pad(h1[:, :, 1:, :], ((0, 0), (0, 0), (0, 1), (0, 0)))
    X2 = jnp.concatenate([h1, h1s], axis=-1).reshape(B * 160, 256)

    z = jnp.dot(X2, w2_ref[...], preferred_element_type=f32)
    z = z.reshape(B, 10, 16, 128)
    a2 = z[:, 0:9, :, 0:64] + z[:, 1:10, :, 64:128]
    a2 = jnp.maximum(a2 + b2_ref[...], 0.0).astype(bf16)

    h2s1 = jnp.pad(a2[:, :, 1:, :], ((0, 0), (0, 0), (0, 1), (0, 0)))
    h2s2 = jnp.pad(a2[:, :, 2:, :], ((0, 0), (0, 0), (0, 2), (0, 0)))
    X3 = jnp.concatenate([a2, h2s1, h2s2], axis=-1).reshape(B * 144, 192)

    v = jnp.dot(X3, w3_ref[...], preferred_element_type=f32)
    v = v.reshape(B, 9, 16, 192)
    a3 = v[:, 0:7, :, 0:64] + v[:, 1:8, :, 64:128] + v[:, 2:9, :, 128:192]
    a3 = jnp.maximum(a3 + b3_ref[...], 0.0)

    o_ref[...] = a3[:, :, 0:7, :]


def _fc_kernel(x_ref, w0_ref, b0_ref, w1_ref, b1_ref, o_ref):
    f32 = jnp.float32
    h = jnp.dot(x_ref[...], w0_ref[...], preferred_element_type=f32)
    h = jnp.maximum(h + b0_ref[...], 0.0).astype(jnp.bfloat16)
    out = jnp.dot(h, w1_ref[...], preferred_element_type=f32)
    o_ref[...] = out + b1_ref[...]


def kernel(x_nchw, conv_taps_0, conv_taps_1, conv_taps_2,
          conv_bias_0, conv_bias_1, conv_bias_2,
          fc_w_0, fc_w_1, fc_b_0, fc_b_1):
    N = x_nchw.shape[0]
    bf16 = jnp.bfloat16
    B = 32

    w1g = conv_taps_0.reshape(2, 4, 2, 4, 4, 128)[..., :32]
    w1g = w1g.transpose(0, 2, 1, 3, 4, 5).reshape(2, 2, 64, 32)
    w1n = jnp.zeros((3, 64, 2, 2, 32), jnp.float32)
    for a in range(2):
        for pw in range(2):
            for q in range(2):
                w1n = w1n.at[pw + q, :, a, pw, :].add(w1g[a, q])
    w1 = w1n.reshape(192, 128).astype(bf16)
    b1c = jnp.concatenate([conv_bias_0[:, :32]] * 2, axis=1)
    w2 = conv_taps_1.reshape(2, 2, 2, 2, 128, 128)[:, :, :, :, :32, :64]
    w2 = w2.transpose(0, 2, 1, 3, 4, 5).reshape(2, 2, 128, 64)
    w2 = jnp.concatenate([w2[:, 0], w2[:, 1]], axis=1)
    w2 = jnp.concatenate([w2[0], w2[1]], axis=1).astype(bf16)
    w3 = conv_taps_2[:, :64, :64].reshape(3, 3, 64, 64)
    w3 = jnp.concatenate([w3[:, 0], w3[:, 1], w3[:, 2]], axis=1)
    w3 = jnp.concatenate([w3[0], w3[1], w3[2]], axis=1).astype(bf16)
    b2 = conv_bias_1[:, :64]
    b3 = conv_bias_2[:, :64]
    fw0 = fc_w_0.reshape(49, 128, 512)[:, :64, :].reshape(3136, 512)
    fw0 = fw0.astype(bf16)
    fw1 = fc_w_1.astype(bf16)

    xs2d = x_nchw.astype(bf16).reshape(N, 4, 21, 4, 21, 4)
    xs2d = xs2d.transpose(0, 2, 4, 3, 5, 1).reshape(N, 21, 21, 64)
    xsp = jnp.pad(xs2d, ((0, 0), (0, 0), (0, 13), (0, 0)))
    pieces = []
    for dw in range(3):
        p = jax.lax.slice(
            xsp, (0, 0, dw, 0), (N, 21, dw + 31, 64), (1, 1, 2, 1))
        pieces.append(p)
    x2 = jnp.concatenate(pieces, axis=-1)
    x2 = x2.reshape(N * 336, 192)

    conv_body = functools.partial(_conv_body, B=B)
    feats = pl.pallas_call(
        conv_body,
        out_shape=jax.ShapeDtypeStruct((N, 7, 7, 64), jnp.float32),
        grid=(N // B,),
        in_specs=[
            pl.BlockSpec((B * 336, 192), lambda n: (n, 0)),
            pl.BlockSpec((192, 128), lambda n: (0, 0)),
            pl.BlockSpec((1, 64), lambda n: (0, 0)),
            pl.BlockSpec((256, 128), lambda n: (0, 0)),
            pl.BlockSpec((1, 64), lambda n: (0, 0)),
            pl.BlockSpec((192, 192), lambda n: (0, 0)),
            pl.BlockSpec((1, 64), lambda n: (0, 0)),
        ],
        out_specs=pl.BlockSpec((B, 7, 7, 64), lambda n: (n, 0, 0, 0)),
        compiler_params=pltpu.CompilerParams(
            dimension_semantics=("parallel",)),
    )(x2, w1, b1c, w2, b2, w3, b3)

    flat = feats.reshape(N, 3136).astype(bf16)
    TM = 256 if N >= 256 else N
    out = pl.pallas_call(
        _fc_kernel,
        out_shape=jax.ShapeDtypeStruct((N, 128), jnp.float32),
        grid=(N // TM,),
        in_specs=[
            pl.BlockSpec((TM, 3136), lambda i: (i, 0)),
            pl.BlockSpec((3136, 512), lambda i: (0, 0)),
            pl.BlockSpec((1, 512), lambda i: (0, 0)),
            pl.BlockSpec((512, 128), lambda i: (0, 0)),
            pl.BlockSpec((1, 128), lambda i: (0, 0)),
        ],
        out_specs=pl.BlockSpec((TM, 128), lambda i: (i, 0)),
        compiler_params=pltpu.CompilerParams(
            dimension_semantics=("parallel",)),
    )(flat, fw0, fc_b_0, fw1, fc_b_1)
    return out[:, :18]
